# baseline (speedup 1.0000x reference)
import functools

import jax
import jax.numpy as jnp
from jax import lax
from jax.experimental import pallas as pl
from jax.experimental.pallas import tpu as pltpu
from jax.experimental.pallas import tpu_sc as plsc

NC = 2
NS = 16
LANES = 16


def _i32(v):
    return jnp.asarray(v, dtype=jnp.int32)


def _dense_body(x_ref, wcat_ref, dvec_ref, asrc_ref, adst_ref,
                dense0_ref, h_ref, as_ref, ad_ref, q_ref):
    xw = jnp.dot(x_ref[...], wcat_ref[...], preferred_element_type=jnp.float32)
    dense0_ref[...] = xw[:, :256] + dvec_ref[...]
    a_s = []
    a_d = []
    for r in range(4):
        hr = xw[:, 256 * (r + 1):256 * (r + 2)]
        h_ref[r, :, :] = hr
        a_s.append(jnp.sum(hr * asrc_ref[r, :][None, :], axis=-1, keepdims=True))
        a_d.append(jnp.sum(hr * adst_ref[r, :][None, :], axis=-1, keepdims=True))
    a_s = jnp.concatenate(a_s, axis=1)
    a_d = jnp.concatenate(a_d, axis=1)
    as_ref[...] = a_s
    ad_ref[...] = a_d
    loop = a_s + a_d
    loop = jnp.maximum(loop, 0.2 * loop)
    q_ref[...] = jnp.exp(loop)


def _dense_stage(x, wcat, dvec, asrc, adst, n):
    bn = 1000
    grid = n // bn
    z = lambda i: (_i32(i), _i32(0))
    zz = lambda i: (_i32(0), _i32(0))
    return pl.pallas_call(
        _dense_body,
        grid=(grid,),
        in_specs=[
            pl.BlockSpec((bn, 256), z),
            pl.BlockSpec((256, 1280), zz),
            pl.BlockSpec((1, 256), zz),
            pl.BlockSpec((4, 256), zz),
            pl.BlockSpec((4, 256), zz),
        ],
        out_specs=[
            pl.BlockSpec((bn, 256), z),
            pl.BlockSpec((4, bn, 256), lambda i: (_i32(0), _i32(i), _i32(0))),
            pl.BlockSpec((bn, 4), z),
            pl.BlockSpec((bn, 4), z),
            pl.BlockSpec((bn, 4), z),
        ],
        out_shape=[
            jax.ShapeDtypeStruct((n, 256), jnp.float32),
            jax.ShapeDtypeStruct((4, n, 256), jnp.float32),
            jax.ShapeDtypeStruct((n, 4), jnp.float32),
            jax.ShapeDtypeStruct((n, 4), jnp.float32),
            jax.ShapeDtypeStruct((n, 4), jnp.float32),
        ],
    )(x, wcat, dvec, asrc, adst)


def _make_sc_stage(n, e):
    ne = 8
    oh = n // ne
    segs = 4 * oh
    srows = (segs + 127) // 128 * 8
    ept = e // NS
    ch = 2000
    nch = ept // ch
    gpc = ch // LANES
    K = 16
    cap = 2560 + 2 * K
    slab = oh // NS
    rem = oh - NS * slab
    assert srows % 80 == 0

    def body(esrc, edst, etyp, as_h, ad_h, q1, dense0, h2, out_sc, den_out,
             tbl, avals, fs_c, dseg_c, p_c, denp, e1, e2, e3,
             rows, wbuf, ridx, den_v, rowidx, qv, out_sh, den_sh, sem):
        c = lax.axis_index("c")
        s = lax.axis_index("s")
        ebase = s * _i32(ept)
        zf = jnp.zeros((LANES,), jnp.float32)
        zi = jnp.zeros((LANES,), jnp.int32)

        pltpu.sync_copy(as_h, tbl)

        def a1_chunk(j, carry):
            off = ebase + j * _i32(ch)
            pltpu.sync_copy(esrc.at[pl.ds(off, ch)], e1)
            pltpu.sync_copy(etyp.at[pl.ds(off, ch)], e3)

            def grp(g, carry2):
                sl = pl.ds(g * _i32(LANES), LANES)
                idx = e1[sl] * _i32(4) + e3[sl]
                avals[pl.ds(j * _i32(ch) + g * _i32(LANES), LANES)] = (
                    plsc.load_gather(tbl, [idx]))
                return carry2
            return lax.fori_loop(_i32(0), _i32(gpc), grp, carry)
        lax.fori_loop(_i32(0), _i32(nch), a1_chunk, _i32(0))

        pltpu.sync_copy(ad_h, tbl)

        def a15_chunk(j, carry):
            off = ebase + j * _i32(ch)
            pltpu.sync_copy(edst.at[pl.ds(off, ch)], e2)
            pltpu.sync_copy(etyp.at[pl.ds(off, ch)], e3)

            def grp(g, carry2):
                sl = pl.ds(g * _i32(LANES), LANES)
                vsl = pl.ds(j * _i32(ch) + g * _i32(LANES), LANES)
                ad16 = plsc.load_gather(tbl, [e2[sl] * _i32(4) + e3[sl]])
                a = avals[vsl] + ad16
                a = jnp.maximum(a, 0.2 * a)
                avals[vsl] = jnp.exp(a)
                return carry2
            return lax.fori_loop(_i32(0), _i32(gpc), grp, carry)
        lax.fori_loop(_i32(0), _i32(nch), a15_chunk, _i32(0))

        for kk in range(srows // 80):
            for g in range(5):
                rowidx[_i32(kk), pl.ds(g * LANES, LANES)] = (
                    lax.iota(jnp.int32, LANES) + _i32(kk * 80 + g * LANES))

        def eighth(qq, carry0):
            e8 = c * _i32(ne // NC) + qq
            qbase = e8 * _i32(oh)

            pltpu.sync_copy(dense0.at[pl.ds(qbase + s * _i32(slab), slab)],
                            out_sh.at[pl.ds(s * _i32(slab), slab)])

            @pl.when(s == 0)
            def _():
                pltpu.sync_copy(dense0.at[pl.ds(qbase + _i32(NS * slab), rem)],
                                out_sh.at[pl.ds(NS * slab, rem)])

            def zero_denp(i, carry):
                denp[i, :] = zf
                return carry
            lax.fori_loop(_i32(0), _i32(srows), zero_denp, _i32(0))

            @pl.when(s == 0)
            def _():
                pltpu.sync_copy(denp, den_sh)

            def a2_chunk(j, cnt):
                off = ebase + j * _i32(ch)
                pltpu.sync_copy(esrc.at[pl.ds(off, ch)], e1)
                pltpu.sync_copy(edst.at[pl.ds(off, ch)], e2)
                pltpu.sync_copy(etyp.at[pl.ds(off, ch)], e3)

                def grp(g, cnt2):
                    sl = pl.ds(g * _i32(LANES), LANES)
                    s16 = e1[sl]
                    d16 = e2[sl]
                    t16 = e3[sl]
                    p = avals[pl.ds(j * _i32(ch) + g * _i32(LANES), LANES)]
                    dloc = d16 - qbase
                    inh = (d16 >= qbase) & (dloc < _i32(oh))
                    dseg = dloc * _i32(4) + t16
                    plsc.addupdate_scatter(
                        denp,
                        [lax.shift_right_logical(dseg, _i32(4)),
                         dseg & _i32(15)], p, mask=inh)
                    plsc.store_compressed(fs_c.at[pl.ds(cnt2, LANES)],
                                          t16 * _i32(n) + s16, mask=inh)
                    plsc.store_compressed(dseg_c.at[pl.ds(cnt2, LANES)], dseg,
                                          mask=inh)
                    plsc.store_compressed(p_c.at[pl.ds(cnt2, LANES)], p, mask=inh)
                    return cnt2 + jnp.sum(inh.astype(jnp.int32), dtype=jnp.int32)
                return lax.fori_loop(_i32(0), _i32(gpc), grp, cnt)
            cnt = lax.fori_loop(_i32(0), _i32(nch), a2_chunk, _i32(0))

            def padk(i, carry):
                fs_c[pl.ds(cnt + i * _i32(LANES), LANES)] = zi
                dseg_c[pl.ds(cnt + i * _i32(LANES), LANES)] = zi
                p_c[pl.ds(cnt + i * _i32(LANES), LANES)] = zf
                return carry
            lax.fori_loop(_i32(0), _i32(2 * K // LANES), padk, _i32(0))

            plsc.subcore_barrier()

            for kk in range(srows // 80):
                pltpu.sync_copy(denp.at[pl.ds(kk * 80, 80)],
                                den_sh.at[rowidx.at[_i32(kk)]], add=True)
            plsc.subcore_barrier()

            @pl.when(s == 1)
            def _():
                pltpu.sync_copy(den_sh, den_out.at[e8])

            pltpu.sync_copy(den_sh, den_v)
            pltpu.sync_copy(q1.at[pl.ds(e8 * _i32(segs), segs)],
                            qv.at[pl.ds(0, segs)])

            def addq(i, carry):
                den_v[i, :] = den_v[i, :] + qv[pl.ds(i * _i32(LANES), LANES)]
                return carry
            lax.fori_loop(_i32(0), _i32((segs + LANES - 1) // LANES), addq,
                          _i32(0))

            nchunks = (cnt + _i32(K - 1)) // _i32(K)

            def issue(ci, rbuf, rsem):
                coff = jnp.minimum(ci, nchunks) * _i32(K)
                pltpu.async_copy(h2.at[fs_c.at[pl.ds(coff, K)]], rbuf, rsem)

            def drain(rbuf, rsem):
                pltpu.make_async_copy(h2.at[pl.ds(0, K)], rbuf, rsem).wait()

            def process(ci, rbuf, rsem):
                coff = ci * _i32(K)

                def g(k, carry2):
                    sl = pl.ds(coff + k * _i32(LANES), LANES)
                    dseg16 = dseg_c[sl]
                    den16 = plsc.load_gather(
                        den_v,
                        [lax.shift_right_logical(dseg16, _i32(4)),
                         dseg16 & _i32(15)])
                    wbuf[pl.ds(k * _i32(LANES), LANES)] = p_c[sl] / den16
                    ridx[pl.ds(k * _i32(LANES), LANES)] = lax.shift_right_logical(
                        dseg16, _i32(2))
                    return carry2
                lax.fori_loop(_i32(0), _i32(K // LANES), g, _i32(0))

                drain(rbuf, rsem)

                def rowscale(r, carry2):
                    wspl = plsc.load_gather(wbuf, [zi + r])
                    for cb in range(256 // LANES):
                        sl = pl.ds(cb * LANES, LANES)
                        rbuf[r, sl] = rbuf[r, sl] * wspl
                    return carry2
                lax.fori_loop(_i32(0), _i32(K), rowscale, _i32(0))

                pltpu.sync_copy(rbuf, out_sh.at[ridx], add=True)

            def one(t, carry):
                issue(t, rows, sem)
                process(t, rows, sem)
                return carry
            lax.fori_loop(_i32(0), nchunks, one, _i32(0))

            plsc.subcore_barrier()

            pltpu.sync_copy(out_sh.at[pl.ds(s * _i32(slab), slab)],
                            out_sc.at[pl.ds(qbase + s * _i32(slab), slab)])

            @pl.when(s == 0)
            def _():
                pltpu.sync_copy(out_sh.at[pl.ds(NS * slab, rem)],
                                out_sc.at[pl.ds(qbase + _i32(NS * slab), rem)])

            plsc.subcore_barrier()
            return carry0
        lax.fori_loop(_i32(0), _i32(ne // NC), eighth, _i32(0))

    mesh = plsc.VectorSubcoreMesh(core_axis_name="c", subcore_axis_name="s",
                                  num_cores=NC, num_subcores=NS)
    return pl.kernel(
        body,
        out_type=[
            jax.ShapeDtypeStruct((n, 256), jnp.float32),
            jax.ShapeDtypeStruct((ne, srows, 16), jnp.float32),
        ],
        mesh=mesh,
        compiler_params=pltpu.CompilerParams(needs_layout_passes=False,
                                             use_tc_tiling_on_sc=False),
        scratch_types=[
            pltpu.VMEM((4 * n,), jnp.float32),
            pltpu.VMEM((ept,), jnp.float32),
            pltpu.VMEM((cap,), jnp.int32),
            pltpu.VMEM((cap,), jnp.int32),
            pltpu.VMEM((cap,), jnp.float32),
            pltpu.VMEM((srows, 16), jnp.float32),
            pltpu.VMEM((ch,), jnp.int32),
            pltpu.VMEM((ch,), jnp.int32),
            pltpu.VMEM((ch,), jnp.int32),
            pltpu.VMEM((K, 256), jnp.float32),
            pltpu.VMEM((K,), jnp.float32),
            pltpu.VMEM((K,), jnp.int32),
            pltpu.VMEM((srows, 16), jnp.float32),
            pltpu.VMEM((srows // 80, 80), jnp.int32),
            pltpu.VMEM((srows * 16,), jnp.float32),
            pltpu.VMEM_SHARED((oh, 256), jnp.float32),
            pltpu.VMEM_SHARED((srows, 16), jnp.float32),
            pltpu.SemaphoreType.DMA,
        ],
    )


def _comb_body(sc_ref, h_ref, q_ref, den_ref, out_ref):
    q = q_ref[...]
    coef = q / (den_ref[...] + q)
    acc = sc_ref[...]
    for r in range(4):
        acc = acc + coef[:, r:r + 1] * h_ref[r]
    out_ref[...] = acc


def _comb_stage(sc_out, h, q, den, n):
    bn = 1000
    grid = n // bn
    z = lambda i: (_i32(i), _i32(0))
    return pl.pallas_call(
        _comb_body,
        grid=(grid,),
        in_specs=[
            pl.BlockSpec((bn, 256), z),
            pl.BlockSpec((4, bn, 256), lambda i: (_i32(0), _i32(i), _i32(0))),
            pl.BlockSpec((bn, 4), z),
            pl.BlockSpec((bn, 4), z),
        ],
        out_specs=pl.BlockSpec((bn, 256), z),
        out_shape=jax.ShapeDtypeStruct((n, 256), jnp.float32),
    )(sc_out, h, q, den)


def kernel(x, edge_index, edge_type, W_self, b_self, W, att_src, att_dst, bias):
    n = x.shape[0]
    e = edge_type.shape[0]
    x = x.astype(jnp.float32)
    wcat = jnp.concatenate(
        [W_self.astype(jnp.float32)] + [W[r].astype(jnp.float32) for r in range(4)],
        axis=1)
    dvec = (b_self + jnp.sum(bias, axis=0)).astype(jnp.float32)[None, :]

    dense0, H, AS, AD, Q = _dense_stage(x, wcat, dvec,
                                        att_src.astype(jnp.float32),
                                        att_dst.astype(jnp.float32), n)

    esrc = edge_index[0].astype(jnp.int32)
    edst = edge_index[1].astype(jnp.int32)
    etyp = edge_type.astype(jnp.int32)

    sc_out, den = _make_sc_stage(n, e)(
        esrc, edst, etyp, AS.reshape(-1), AD.reshape(-1), Q.reshape(-1),
        dense0, H.reshape(4 * n, 256))

    segs = 4 * n // 8
    den = den.reshape(8, -1)[:, :segs].reshape(n, 4)
    out = _comb_stage(sc_out, H, Q, den, n)
    return out.astype(jnp.float64)


# speedup vs baseline: 1.0702x; 1.0702x over previous
import jax
import jax.numpy as jnp
from jax import lax
from jax.experimental import pallas as pl
from jax.experimental.pallas import tpu as pltpu
from jax.experimental.pallas import tpu_sc as plsc

NC = 2
NS = 16
LANES = 16


def _i32(v):
    return jnp.asarray(v, dtype=jnp.int32)


def _dense_body(x_ref, wcat_ref, dvec_ref, asrc_ref, adst_ref,
                dense0_ref, h_ref, as_ref, ad_ref, q_ref):
    xw = jnp.dot(x_ref[...], wcat_ref[...], preferred_element_type=jnp.float32)
    dense0_ref[...] = xw[:, :256] + dvec_ref[...]
    a_s = []
    a_d = []
    for r in range(4):
        hr = xw[:, 256 * (r + 1):256 * (r + 2)]
        h_ref[r, :, :] = hr
        a_s.append(jnp.sum(hr * asrc_ref[r, :][None, :], axis=-1, keepdims=True))
        a_d.append(jnp.sum(hr * adst_ref[r, :][None, :], axis=-1, keepdims=True))
    a_s = jnp.concatenate(a_s, axis=1)
    a_d = jnp.concatenate(a_d, axis=1)
    as_ref[...] = a_s
    ad_ref[...] = a_d
    loop = a_s + a_d
    loop = jnp.maximum(loop, 0.2 * loop)
    q_ref[...] = jnp.exp(loop)


def _dense_stage(x, wcat, dvec, asrc, adst, n):
    bn = 1000
    grid = n // bn
    z = lambda i: (_i32(i), _i32(0))
    zz = lambda i: (_i32(0), _i32(0))
    return pl.pallas_call(
        _dense_body,
        grid=(grid,),
        in_specs=[
            pl.BlockSpec((bn, 256), z),
            pl.BlockSpec((256, 1280), zz),
            pl.BlockSpec((1, 256), zz),
            pl.BlockSpec((4, 256), zz),
            pl.BlockSpec((4, 256), zz),
        ],
        out_specs=[
            pl.BlockSpec((bn, 256), z),
            pl.BlockSpec((4, bn, 256), lambda i: (_i32(0), _i32(i), _i32(0))),
            pl.BlockSpec((bn, 4), z),
            pl.BlockSpec((bn, 4), z),
            pl.BlockSpec((bn, 4), z),
        ],
        out_shape=[
            jax.ShapeDtypeStruct((n, 256), jnp.float32),
            jax.ShapeDtypeStruct((4, n, 256), jnp.float32),
            jax.ShapeDtypeStruct((n, 4), jnp.float32),
            jax.ShapeDtypeStruct((n, 4), jnp.float32),
            jax.ShapeDtypeStruct((n, 4), jnp.float32),
        ],
    )(x, wcat, dvec, asrc, adst)


def _make_sc_stage(n, e):
    ne = 8
    oh = n // ne
    segs = 4 * oh
    srows = (segs + 127) // 128 * 8
    ept = e // NS
    ch = 2000
    nch = ept // ch
    gpc = ch // LANES
    K = 32
    cap = 2560 + 2 * K
    slab = oh // NS
    rem = oh - NS * slab
    assert srows % 80 == 0

    def body(esrc, edst, etyp, as_h, ad_h, q1, dense0, h2, out_sc, den_out,
             tbl, avals, fs_c, dseg_c, p_c, denp, e1, e2, e3,
             rows, wbuf, ridx, den_v, rowidx, qv, out_sh, den_sh, sem):
        c = lax.axis_index("c")
        s = lax.axis_index("s")
        ebase = s * _i32(ept)
        zf = jnp.zeros((LANES,), jnp.float32)
        zi = jnp.zeros((LANES,), jnp.int32)

        pltpu.sync_copy(as_h, tbl)

        def a1_chunk(j, carry):
            off = ebase + j * _i32(ch)
            pltpu.sync_copy(esrc.at[pl.ds(off, ch)], e1)
            pltpu.sync_copy(etyp.at[pl.ds(off, ch)], e3)

            def grp(g, carry2):
                sl = pl.ds(g * _i32(LANES), LANES)
                idx = e1[sl] * _i32(4) + e3[sl]
                avals[pl.ds(j * _i32(ch) + g * _i32(LANES), LANES)] = (
                    plsc.load_gather(tbl, [idx]))
                return carry2
            return lax.fori_loop(_i32(0), _i32(gpc), grp, carry)
        lax.fori_loop(_i32(0), _i32(nch), a1_chunk, _i32(0))

        pltpu.sync_copy(ad_h, tbl)

        def a15_chunk(j, carry):
            off = ebase + j * _i32(ch)
            pltpu.sync_copy(edst.at[pl.ds(off, ch)], e2)
            pltpu.sync_copy(etyp.at[pl.ds(off, ch)], e3)

            def grp(g, carry2):
                sl = pl.ds(g * _i32(LANES), LANES)
                vsl = pl.ds(j * _i32(ch) + g * _i32(LANES), LANES)
                ad16 = plsc.load_gather(tbl, [e2[sl] * _i32(4) + e3[sl]])
                a = avals[vsl] + ad16
                a = jnp.maximum(a, 0.2 * a)
                avals[vsl] = jnp.exp(a)
                return carry2
            return lax.fori_loop(_i32(0), _i32(gpc), grp, carry)
        lax.fori_loop(_i32(0), _i32(nch), a15_chunk, _i32(0))

        for kk in range(srows // 80):
            for g in range(5):
                rowidx[_i32(kk), pl.ds(g * LANES, LANES)] = (
                    lax.iota(jnp.int32, LANES) + _i32(kk * 80 + g * LANES))

        def eighth(qq, carry0):
            e8 = c * _i32(ne // NC) + qq
            qbase = e8 * _i32(oh)

            pltpu.sync_copy(dense0.at[pl.ds(qbase + s * _i32(slab), slab)],
                            out_sh.at[pl.ds(s * _i32(slab), slab)])

            @pl.when(s == 0)
            def _():
                pltpu.sync_copy(dense0.at[pl.ds(qbase + _i32(NS * slab), rem)],
                                out_sh.at[pl.ds(NS * slab, rem)])

            def zero_denp(i, carry):
                denp[i, :] = zf
                return carry
            lax.fori_loop(_i32(0), _i32(srows), zero_denp, _i32(0))

            @pl.when(s == 0)
            def _():
                pltpu.sync_copy(denp, den_sh)

            def a2_chunk(j, cnt):
                off = ebase + j * _i32(ch)
                pltpu.sync_copy(esrc.at[pl.ds(off, ch)], e1)
                pltpu.sync_copy(edst.at[pl.ds(off, ch)], e2)
                pltpu.sync_copy(etyp.at[pl.ds(off, ch)], e3)

                def grp(g, cnt2):
                    sl = pl.ds(g * _i32(LANES), LANES)
                    s16 = e1[sl]
                    d16 = e2[sl]
                    t16 = e3[sl]
                    p = avals[pl.ds(j * _i32(ch) + g * _i32(LANES), LANES)]
                    dloc = d16 - qbase
                    inh = (d16 >= qbase) & (dloc < _i32(oh))
                    dseg = dloc * _i32(4) + t16
                    plsc.addupdate_scatter(
                        denp,
                        [lax.shift_right_logical(dseg, _i32(4)),
                         dseg & _i32(15)], p, mask=inh)
                    plsc.store_compressed(fs_c.at[pl.ds(cnt2, LANES)],
                                          t16 * _i32(n) + s16, mask=inh)
                    plsc.store_compressed(dseg_c.at[pl.ds(cnt2, LANES)], dseg,
                                          mask=inh)
                    plsc.store_compressed(p_c.at[pl.ds(cnt2, LANES)], p, mask=inh)
                    return cnt2 + jnp.sum(inh.astype(jnp.int32), dtype=jnp.int32)
                return lax.fori_loop(_i32(0), _i32(gpc), grp, cnt)
            cnt = lax.fori_loop(_i32(0), _i32(nch), a2_chunk, _i32(0))

            def padk(i, carry):
                fs_c[pl.ds(cnt + i * _i32(LANES), LANES)] = zi
                dseg_c[pl.ds(cnt + i * _i32(LANES), LANES)] = zi
                p_c[pl.ds(cnt + i * _i32(LANES), LANES)] = zf
                return carry
            lax.fori_loop(_i32(0), _i32(2 * K // LANES), padk, _i32(0))

            plsc.subcore_barrier()

            for kk in range(srows // 80):
                pltpu.sync_copy(denp.at[pl.ds(kk * 80, 80)],
                                den_sh.at[rowidx.at[_i32(kk)]], add=True)
            plsc.subcore_barrier()

            @pl.when(s == 1)
            def _():
                pltpu.sync_copy(den_sh, den_out.at[e8])

            pltpu.sync_copy(den_sh, den_v)
            pltpu.sync_copy(q1.at[pl.ds(e8 * _i32(segs), segs)],
                            qv.at[pl.ds(0, segs)])

            def addq(i, carry):
                den_v[i, :] = den_v[i, :] + qv[pl.ds(i * _i32(LANES), LANES)]
                return carry
            lax.fori_loop(_i32(0), _i32((segs + LANES - 1) // LANES), addq,
                          _i32(0))

            nchunks = (cnt + _i32(K - 1)) // _i32(K)

            def issue(ci, rbuf, rsem):
                coff = jnp.minimum(ci, nchunks) * _i32(K)
                pltpu.async_copy(h2.at[fs_c.at[pl.ds(coff, K)]], rbuf, rsem)

            def drain(rbuf, rsem):
                pltpu.make_async_copy(h2.at[pl.ds(0, K)], rbuf, rsem).wait()

            def process(ci, rbuf, rsem):
                coff = ci * _i32(K)

                def g(k, carry2):
                    sl = pl.ds(coff + k * _i32(LANES), LANES)
                    dseg16 = dseg_c[sl]
                    den16 = plsc.load_gather(
                        den_v,
                        [lax.shift_right_logical(dseg16, _i32(4)),
                         dseg16 & _i32(15)])
                    wbuf[pl.ds(k * _i32(LANES), LANES)] = p_c[sl] / den16
                    ridx[pl.ds(k * _i32(LANES), LANES)] = lax.shift_right_logical(
                        dseg16, _i32(2))
                    return carry2
                lax.fori_loop(_i32(0), _i32(K // LANES), g, _i32(0))

                drain(rbuf, rsem)

                def rowscale(r, carry2):
                    wspl = plsc.load_gather(wbuf, [zi + r])
                    for cb in range(256 // LANES):
                        sl = pl.ds(cb * LANES, LANES)
                        rbuf[r, sl] = rbuf[r, sl] * wspl
                    return carry2
                lax.fori_loop(_i32(0), _i32(K), rowscale, _i32(0))

                pltpu.sync_copy(rbuf, out_sh.at[ridx], add=True)

            def one(t, carry):
                issue(t, rows, sem)
                process(t, rows, sem)
                return carry
            lax.fori_loop(_i32(0), nchunks, one, _i32(0))

            plsc.subcore_barrier()

            pltpu.sync_copy(out_sh.at[pl.ds(s * _i32(slab), slab)],
                            out_sc.at[pl.ds(qbase + s * _i32(slab), slab)])

            @pl.when(s == 0)
            def _():
                pltpu.sync_copy(out_sh.at[pl.ds(NS * slab, rem)],
                                out_sc.at[pl.ds(qbase + _i32(NS * slab), rem)])

            plsc.subcore_barrier()
            return carry0
        lax.fori_loop(_i32(0), _i32(ne // NC), eighth, _i32(0))

    mesh = plsc.VectorSubcoreMesh(core_axis_name="c", subcore_axis_name="s",
                                  num_cores=NC, num_subcores=NS)
    return pl.kernel(
        body,
        out_type=[
            jax.ShapeDtypeStruct((n, 256), jnp.float32),
            jax.ShapeDtypeStruct((ne, srows, 16), jnp.float32),
        ],
        mesh=mesh,
        compiler_params=pltpu.CompilerParams(needs_layout_passes=False,
                                             use_tc_tiling_on_sc=False),
        scratch_types=[
            pltpu.VMEM((4 * n,), jnp.float32),
            pltpu.VMEM((ept,), jnp.float32),
            pltpu.VMEM((cap,), jnp.int32),
            pltpu.VMEM((cap,), jnp.int32),
            pltpu.VMEM((cap,), jnp.float32),
            pltpu.VMEM((srows, 16), jnp.float32),
            pltpu.VMEM((ch,), jnp.int32),
            pltpu.VMEM((ch,), jnp.int32),
            pltpu.VMEM((ch,), jnp.int32),
            pltpu.VMEM((K, 256), jnp.float32),
            pltpu.VMEM((K,), jnp.float32),
            pltpu.VMEM((K,), jnp.int32),
            pltpu.VMEM((srows, 16), jnp.float32),
            pltpu.VMEM((srows // 80, 80), jnp.int32),
            pltpu.VMEM((srows * 16,), jnp.float32),
            pltpu.VMEM_SHARED((oh, 256), jnp.float32),
            pltpu.VMEM_SHARED((srows, 16), jnp.float32),
            pltpu.SemaphoreType.DMA,
        ],
    )


def _comb_body(sc_ref, h_ref, q_ref, den_ref, out_ref):
    q = q_ref[...]
    coef = q / (den_ref[...] + q)
    acc = sc_ref[...]
    for r in range(4):
        acc = acc + coef[:, r:r + 1] * h_ref[r]
    out_ref[...] = acc


def _comb_stage(sc_out, h, q, den, n):
    bn = 1000
    grid = n // bn
    z = lambda i: (_i32(i), _i32(0))
    return pl.pallas_call(
        _comb_body,
        grid=(grid,),
        in_specs=[
            pl.BlockSpec((bn, 256), z),
            pl.BlockSpec((4, bn, 256), lambda i: (_i32(0), _i32(i), _i32(0))),
            pl.BlockSpec((bn, 4), z),
            pl.BlockSpec((bn, 4), z),
        ],
        out_specs=pl.BlockSpec((bn, 256), z),
        out_shape=jax.ShapeDtypeStruct((n, 256), jnp.float32),
    )(sc_out, h, q, den)


def kernel(x, edge_index, edge_type, W_self, b_self, W, att_src, att_dst, bias):
    n = x.shape[0]
    e = edge_type.shape[0]
    x = x.astype(jnp.float32)
    wcat = jnp.concatenate(
        [W_self.astype(jnp.float32)] + [W[r].astype(jnp.float32) for r in range(4)],
        axis=1)
    dvec = (b_self + jnp.sum(bias, axis=0)).astype(jnp.float32)[None, :]

    dense0, H, AS, AD, Q = _dense_stage(x, wcat, dvec,
                                        att_src.astype(jnp.float32),
                                        att_dst.astype(jnp.float32), n)

    esrc = edge_index[0].astype(jnp.int32)
    edst = edge_index[1].astype(jnp.int32)
    etyp = edge_type.astype(jnp.int32)

    sc_out, den = _make_sc_stage(n, e)(
        esrc, edst, etyp, AS.reshape(-1), AD.reshape(-1), Q.reshape(-1),
        dense0, H.reshape(4 * n, 256))

    segs = 4 * n // 8
    den = den.reshape(8, -1)[:, :segs].reshape(n, 4)
    out = _comb_stage(sc_out, H, Q, den, n)
    return out.astype(jnp.float64)


# speedup vs baseline: 1.1097x; 1.0369x over previous
import jax
import jax.numpy as jnp
from jax import lax
from jax.experimental import pallas as pl
from jax.experimental.pallas import tpu as pltpu
from jax.experimental.pallas import tpu_sc as plsc

NC = 2
NS = 16
LANES = 16


def _i32(v):
    return jnp.asarray(v, dtype=jnp.int32)


def _dense_body(x_ref, wcat_ref, dvec_ref, asrc_ref, adst_ref,
                dense0_ref, h_ref, as_ref, ad_ref, q_ref):
    xw = jnp.dot(x_ref[...], wcat_ref[...], preferred_element_type=jnp.float32)
    dense0_ref[...] = xw[:, :256] + dvec_ref[...]
    a_s = []
    a_d = []
    for r in range(4):
        hr = xw[:, 256 * (r + 1):256 * (r + 2)]
        h_ref[r, :, :] = hr
        a_s.append(jnp.sum(hr * asrc_ref[r, :][None, :], axis=-1, keepdims=True))
        a_d.append(jnp.sum(hr * adst_ref[r, :][None, :], axis=-1, keepdims=True))
    a_s = jnp.concatenate(a_s, axis=1)
    a_d = jnp.concatenate(a_d, axis=1)
    as_ref[...] = a_s
    ad_ref[...] = a_d
    loop = a_s + a_d
    loop = jnp.maximum(loop, 0.2 * loop)
    q_ref[...] = jnp.exp(loop)


def _dense_stage(x, wcat, dvec, asrc, adst, n):
    bn = 1000
    grid = n // bn
    z = lambda i: (_i32(i), _i32(0))
    zz = lambda i: (_i32(0), _i32(0))
    return pl.pallas_call(
        _dense_body,
        grid=(grid,),
        in_specs=[
            pl.BlockSpec((bn, 256), z),
            pl.BlockSpec((256, 1280), zz),
            pl.BlockSpec((1, 256), zz),
            pl.BlockSpec((4, 256), zz),
            pl.BlockSpec((4, 256), zz),
        ],
        out_specs=[
            pl.BlockSpec((bn, 256), z),
            pl.BlockSpec((4, bn, 256), lambda i: (_i32(0), _i32(i), _i32(0))),
            pl.BlockSpec((bn, 4), z),
            pl.BlockSpec((bn, 4), z),
            pl.BlockSpec((bn, 4), z),
        ],
        out_shape=[
            jax.ShapeDtypeStruct((n, 256), jnp.float32),
            jax.ShapeDtypeStruct((4, n, 256), jnp.float32),
            jax.ShapeDtypeStruct((n, 4), jnp.float32),
            jax.ShapeDtypeStruct((n, 4), jnp.float32),
            jax.ShapeDtypeStruct((n, 4), jnp.float32),
        ],
    )(x, wcat, dvec, asrc, adst)


def _make_sc_stage(n, e):
    ne = 8
    oh = n // ne
    segs = 4 * oh
    srows = (segs + 127) // 128 * 8
    ept = e // NS
    ch = 2000
    nch = ept // ch
    gpc = ch // LANES
    K = 32
    cap = 2560 + 2 * K
    slab = oh // NS
    rem = oh - NS * slab
    assert srows % 80 == 0

    def body(epk, as_h, ad_h, q1, dense0, h2, out_sc, den_out,
             tbl, avals, fs_c, dseg_c, p_c, denp, e123,
             rows, wbuf, ridx, den_v, rowidx, qv, out_sh, den_sh, sem):
        c = lax.axis_index("c")
        s = lax.axis_index("s")
        zf = jnp.zeros((LANES,), jnp.float32)
        zi = jnp.zeros((LANES,), jnp.int32)

        pltpu.sync_copy(as_h, tbl)

        def a1_chunk(j, carry):
            pltpu.sync_copy(epk.at[s * _i32(nch) + j], e123)

            def grp(g, carry2):
                sl = pl.ds(g * _i32(LANES), LANES)
                idx = e123[0, sl] * _i32(4) + e123[2, sl]
                avals[pl.ds(j * _i32(ch) + g * _i32(LANES), LANES)] = (
                    plsc.load_gather(tbl, [idx]))
                return carry2
            return lax.fori_loop(_i32(0), _i32(gpc), grp, carry)
        lax.fori_loop(_i32(0), _i32(nch), a1_chunk, _i32(0))

        pltpu.sync_copy(ad_h, tbl)

        def a15_chunk(j, carry):
            pltpu.sync_copy(epk.at[s * _i32(nch) + j], e123)

            def grp(g, carry2):
                sl = pl.ds(g * _i32(LANES), LANES)
                vsl = pl.ds(j * _i32(ch) + g * _i32(LANES), LANES)
                ad16 = plsc.load_gather(tbl,
                                        [e123[1, sl] * _i32(4) + e123[2, sl]])
                a = avals[vsl] + ad16
                a = jnp.maximum(a, 0.2 * a)
                avals[vsl] = jnp.exp(a)
                return carry2
            return lax.fori_loop(_i32(0), _i32(gpc), grp, carry)
        lax.fori_loop(_i32(0), _i32(nch), a15_chunk, _i32(0))

        for kk in range(srows // 80):
            for g in range(5):
                rowidx[_i32(kk), pl.ds(g * LANES, LANES)] = (
                    lax.iota(jnp.int32, LANES) + _i32(kk * 80 + g * LANES))

        def eighth(qq, carry0):
            e8 = c * _i32(ne // NC) + qq
            qbase = e8 * _i32(oh)

            pltpu.sync_copy(dense0.at[pl.ds(qbase + s * _i32(slab), slab)],
                            out_sh.at[pl.ds(s * _i32(slab), slab)])

            @pl.when(s == 0)
            def _():
                pltpu.sync_copy(dense0.at[pl.ds(qbase + _i32(NS * slab), rem)],
                                out_sh.at[pl.ds(NS * slab, rem)])

            def zero_denp(i, carry):
                denp[i, :] = zf
                return carry
            lax.fori_loop(_i32(0), _i32(srows), zero_denp, _i32(0))

            @pl.when(s == 0)
            def _():
                pltpu.sync_copy(denp, den_sh)

            def a2_chunk(j, cnt):
                pltpu.sync_copy(epk.at[s * _i32(nch) + j], e123)

                def grp(g, cnt2):
                    sl = pl.ds(g * _i32(LANES), LANES)
                    s16 = e123[0, sl]
                    d16 = e123[1, sl]
                    t16 = e123[2, sl]
                    p = avals[pl.ds(j * _i32(ch) + g * _i32(LANES), LANES)]
                    dloc = d16 - qbase
                    inh = (d16 >= qbase) & (dloc < _i32(oh))
                    dseg = dloc * _i32(4) + t16
                    plsc.addupdate_scatter(
                        denp,
                        [lax.shift_right_logical(dseg, _i32(4)),
                         dseg & _i32(15)], p, mask=inh)
                    plsc.store_compressed(fs_c.at[pl.ds(cnt2, LANES)],
                                          t16 * _i32(n) + s16, mask=inh)
                    plsc.store_compressed(dseg_c.at[pl.ds(cnt2, LANES)], dseg,
                                          mask=inh)
                    plsc.store_compressed(p_c.at[pl.ds(cnt2, LANES)], p, mask=inh)
                    return cnt2 + jnp.sum(inh.astype(jnp.int32), dtype=jnp.int32)
                return lax.fori_loop(_i32(0), _i32(gpc), grp, cnt)
            cnt = lax.fori_loop(_i32(0), _i32(nch), a2_chunk, _i32(0))

            def padk(i, carry):
                fs_c[pl.ds(cnt + i * _i32(LANES), LANES)] = zi
                dseg_c[pl.ds(cnt + i * _i32(LANES), LANES)] = zi
                p_c[pl.ds(cnt + i * _i32(LANES), LANES)] = zf
                return carry
            lax.fori_loop(_i32(0), _i32(2 * K // LANES), padk, _i32(0))

            plsc.subcore_barrier()

            for kk in range(srows // 80):
                pltpu.sync_copy(denp.at[pl.ds(kk * 80, 80)],
                                den_sh.at[rowidx.at[_i32(kk)]], add=True)
            plsc.subcore_barrier()

            @pl.when(s == 1)
            def _():
                pltpu.sync_copy(den_sh, den_out.at[e8])

            pltpu.sync_copy(den_sh, den_v)
            pltpu.sync_copy(q1.at[pl.ds(e8 * _i32(segs), segs)],
                            qv.at[pl.ds(0, segs)])

            def addq(i, carry):
                den_v[i, :] = den_v[i, :] + qv[pl.ds(i * _i32(LANES), LANES)]
                return carry
            lax.fori_loop(_i32(0), _i32((segs + LANES - 1) // LANES), addq,
                          _i32(0))

            nchunks = (cnt + _i32(K - 1)) // _i32(K)

            def issue(ci, rbuf, rsem):
                coff = jnp.minimum(ci, nchunks) * _i32(K)
                pltpu.async_copy(h2.at[fs_c.at[pl.ds(coff, K)]], rbuf, rsem)

            def drain(rbuf, rsem):
                pltpu.make_async_copy(h2.at[pl.ds(0, K)], rbuf, rsem).wait()

            def process(ci, rbuf, rsem):
                coff = ci * _i32(K)

                def g(k, carry2):
                    sl = pl.ds(coff + k * _i32(LANES), LANES)
                    dseg16 = dseg_c[sl]
                    den16 = plsc.load_gather(
                        den_v,
                        [lax.shift_right_logical(dseg16, _i32(4)),
                         dseg16 & _i32(15)])
                    wbuf[pl.ds(k * _i32(LANES), LANES)] = p_c[sl] / den16
                    ridx[pl.ds(k * _i32(LANES), LANES)] = lax.shift_right_logical(
                        dseg16, _i32(2))
                    return carry2
                lax.fori_loop(_i32(0), _i32(K // LANES), g, _i32(0))

                drain(rbuf, rsem)

                def rowscale(r, carry2):
                    wspl = plsc.load_gather(wbuf, [zi + r])
                    for cb in range(256 // LANES):
                        sl = pl.ds(cb * LANES, LANES)
                        rbuf[r, sl] = rbuf[r, sl] * wspl
                    return carry2
                lax.fori_loop(_i32(0), _i32(K), rowscale, _i32(0))

                pltpu.sync_copy(rbuf, out_sh.at[ridx], add=True)

            def one(t, carry):
                issue(t, rows, sem)
                process(t, rows, sem)
                return carry
            lax.fori_loop(_i32(0), nchunks, one, _i32(0))

            plsc.subcore_barrier()

            pltpu.sync_copy(out_sh.at[pl.ds(s * _i32(slab), slab)],
                            out_sc.at[pl.ds(qbase + s * _i32(slab), slab)])

            @pl.when(s == 0)
            def _():
                pltpu.sync_copy(out_sh.at[pl.ds(NS * slab, rem)],
                                out_sc.at[pl.ds(qbase + _i32(NS * slab), rem)])

            plsc.subcore_barrier()
            return carry0
        lax.fori_loop(_i32(0), _i32(ne // NC), eighth, _i32(0))

    mesh = plsc.VectorSubcoreMesh(core_axis_name="c", subcore_axis_name="s",
                                  num_cores=NC, num_subcores=NS)
    return pl.kernel(
        body,
        out_type=[
            jax.ShapeDtypeStruct((n, 256), jnp.float32),
            jax.ShapeDtypeStruct((ne, srows, 16), jnp.float32),
        ],
        mesh=mesh,
        compiler_params=pltpu.CompilerParams(needs_layout_passes=False,
                                             use_tc_tiling_on_sc=False),
        scratch_types=[
            pltpu.VMEM((4 * n,), jnp.float32),
            pltpu.VMEM((ept,), jnp.float32),
            pltpu.VMEM((cap,), jnp.int32),
            pltpu.VMEM((cap,), jnp.int32),
            pltpu.VMEM((cap,), jnp.float32),
            pltpu.VMEM((srows, 16), jnp.float32),
            pltpu.VMEM((3, ch), jnp.int32),
            pltpu.VMEM((K, 256), jnp.float32),
            pltpu.VMEM((K,), jnp.float32),
            pltpu.VMEM((K,), jnp.int32),
            pltpu.VMEM((srows, 16), jnp.float32),
            pltpu.VMEM((srows // 80, 80), jnp.int32),
            pltpu.VMEM((srows * 16,), jnp.float32),
            pltpu.VMEM_SHARED((oh, 256), jnp.float32),
            pltpu.VMEM_SHARED((srows, 16), jnp.float32),
            pltpu.SemaphoreType.DMA,
        ],
    )


def _comb_body(sc_ref, h_ref, q_ref, den_ref, out_ref):
    q = q_ref[...]
    coef = q / (den_ref[...] + q)
    acc = sc_ref[...]
    for r in range(4):
        acc = acc + coef[:, r:r + 1] * h_ref[r]
    out_ref[...] = acc


def _comb_stage(sc_out, h, q, den, n):
    bn = 1000
    grid = n // bn
    z = lambda i: (_i32(i), _i32(0))
    return pl.pallas_call(
        _comb_body,
        grid=(grid,),
        in_specs=[
            pl.BlockSpec((bn, 256), z),
            pl.BlockSpec((4, bn, 256), lambda i: (_i32(0), _i32(i), _i32(0))),
            pl.BlockSpec((bn, 4), z),
            pl.BlockSpec((bn, 4), z),
        ],
        out_specs=pl.BlockSpec((bn, 256), z),
        out_shape=jax.ShapeDtypeStruct((n, 256), jnp.float32),
    )(sc_out, h, q, den)


def kernel(x, edge_index, edge_type, W_self, b_self, W, att_src, att_dst, bias):
    n = x.shape[0]
    e = edge_type.shape[0]
    x = x.astype(jnp.float32)
    wcat = jnp.concatenate(
        [W_self.astype(jnp.float32)] + [W[r].astype(jnp.float32) for r in range(4)],
        axis=1)
    dvec = (b_self + jnp.sum(bias, axis=0)).astype(jnp.float32)[None, :]

    dense0, H, AS, AD, Q = _dense_stage(x, wcat, dvec,
                                        att_src.astype(jnp.float32),
                                        att_dst.astype(jnp.float32), n)

    ch = 2000
    epk = jnp.concatenate(
        [edge_index.astype(jnp.int32), edge_type.astype(jnp.int32)[None, :]],
        axis=0)
    epk = epk.reshape(3, e // ch, ch).transpose(1, 0, 2)

    sc_out, den = _make_sc_stage(n, e)(
        epk, AS.reshape(-1), AD.reshape(-1), Q.reshape(-1),
        dense0, H.reshape(4 * n, 256))

    segs = 4 * n // 8
    den = den.reshape(8, -1)[:, :segs].reshape(n, 4)
    out = _comb_stage(sc_out, H, Q, den, n)
    return out.astype(jnp.float64)
